# baseline (speedup 1.0000x reference)
import functools

import jax
import jax.numpy as jnp
from jax import lax
from jax.experimental import pallas as pl
from jax.experimental.pallas import tpu as pltpu
from jax.experimental.pallas import tpu_sc as plsc

NC = 2
NS = 16
NW = NC * NS
LANES = 16


_PACK = 8


def _edge_mlp_body(ef_ref, w0_ref, b0_ref, w1_ref, b1_ref, w2_ref, b2_ref,
                   e0_ref, e1_ref, e2_ref):
    ef = ef_ref[...]
    e0_ref[...] = jnp.dot(ef, w0_ref[...],
                          preferred_element_type=jnp.float32) + b0_ref[...]
    e1_ref[...] = jnp.dot(ef, w1_ref[...],
                          preferred_element_type=jnp.float32) + b1_ref[...]
    e2_ref[...] = jnp.dot(ef, w2_ref[...],
                          preferred_element_type=jnp.float32) + b2_ref[...]


def _edge_mlp3(ef, wb):
    E, ED = ef.shape
    D = wb[0][0].shape[1]
    K = _PACK * ED
    ND = _PACK * D
    ef8 = ef.reshape(E // _PACK, K)
    packed = []
    for we, be in wb:
        w8 = jnp.zeros((_PACK, ED, _PACK, D), jnp.float32)
        for i in range(_PACK):
            w8 = w8.at[i, :, i, :].set(we)
        packed.append((w8.reshape(K, ND), jnp.tile(be, _PACK)))
    BE = 8000 // _PACK
    grid = ef8.shape[0] // BE
    full = lambda i: (0, 0)
    vec = lambda i: (0,)
    out = pl.pallas_call(
        _edge_mlp_body,
        grid=(grid,),
        in_specs=[
            pl.BlockSpec((BE, K), lambda i: (i, 0)),
            pl.BlockSpec((K, ND), full), pl.BlockSpec((ND,), vec),
            pl.BlockSpec((K, ND), full), pl.BlockSpec((ND,), vec),
            pl.BlockSpec((K, ND), full), pl.BlockSpec((ND,), vec),
        ],
        out_specs=[pl.BlockSpec((BE, ND), lambda i: (i, 0))] * 3,
        out_shape=[jax.ShapeDtypeStruct((E // _PACK, ND), jnp.float32)] * 3,
    )(ef8, packed[0][0], packed[0][1], packed[1][0], packed[1][1],
      packed[2][0], packed[2][1])
    return [e.reshape(E, D) for e in out]


_EDGE_CHUNK = 80


def _make_edge_agg(N, D, E):
    e_per_w = E // NW
    n_chunks = e_per_w // _EDGE_CHUNK
    rows_main = (N // NS) // 8 * 8
    rows_tail = N - rows_main * NS
    mesh = plsc.VectorSubcoreMesh(core_axis_name="c", subcore_axis_name="s",
                                  num_cores=NC, num_subcores=NS)
    zcopies = rows_main // _EDGE_CHUNK
    ztail = rows_main - zcopies * _EDGE_CHUNK

    assert n_chunks % 2 == 1

    @functools.partial(
        pl.kernel,
        out_type=jax.ShapeDtypeStruct((NC, N, D), jnp.float32),
        mesh=mesh,
        scratch_types=[
            pltpu.VMEM((_EDGE_CHUNK,), jnp.int32),
            pltpu.VMEM((_EDGE_CHUNK,), jnp.int32),
            pltpu.VMEM((_EDGE_CHUNK,), jnp.int32),
            pltpu.VMEM((_EDGE_CHUNK,), jnp.int32),
            pltpu.VMEM((_EDGE_CHUNK, D), jnp.float32),
            pltpu.VMEM((_EDGE_CHUNK, D), jnp.float32),
            pltpu.VMEM_SHARED((N, D), jnp.float32),
            pltpu.SemaphoreType.DMA,
            pltpu.SemaphoreType.DMA,
            pltpu.SemaphoreType.DMA,
            pltpu.SemaphoreType.DMA,
            pltpu.SemaphoreType.DMA,
            pltpu.SemaphoreType.DMA,
        ],
    )
    def edge_agg(h_hbm, e_hbm, src_hbm, dst_hbm, out_hbm,
                 src_a, src_b, dst_a, dst_b, buf_a, buf_b, acc_sh,
                 semi_a, semi_b, seme_a, seme_b, semg_a, semg_b):
        c = lax.axis_index("c")
        s = lax.axis_index("s")
        wid = c * NS + s
        base = wid * e_per_w
        row0 = s * rows_main
        srcs = (src_a, src_b)
        dsts = (dst_a, dst_b)
        bufs = (buf_a, buf_b)
        semi = (semi_a, semi_b)
        seme = (seme_a, seme_b)
        semg = (semg_a, semg_b)

        def stage_issue(j, b):
            off = base + j * _EDGE_CHUNK
            pltpu.async_copy(src_hbm.at[pl.ds(off, _EDGE_CHUNK)],
                             srcs[b], semi[b])
            pltpu.async_copy(dst_hbm.at[pl.ds(off, _EDGE_CHUNK)],
                             dsts[b], semi[b])
            pltpu.async_copy(e_hbm.at[pl.ds(off, _EDGE_CHUNK)],
                             bufs[b], seme[b])

        def stage_gather(b):
            pltpu.make_async_copy(src_hbm.at[pl.ds(0, _EDGE_CHUNK)],
                                  srcs[b], semi[b]).wait()
            pltpu.make_async_copy(dst_hbm.at[pl.ds(0, _EDGE_CHUNK)],
                                  dsts[b], semi[b]).wait()
            pltpu.make_async_copy(e_hbm.at[pl.ds(0, _EDGE_CHUNK)],
                                  bufs[b], seme[b]).wait()
            pltpu.async_copy(h_hbm.at[srcs[b]], bufs[b], semg[b], add=True)

        def stage_reduce(b):
            pltpu.make_async_copy(h_hbm.at[srcs[b]], bufs[b], semg[b]).wait()
            buf = bufs[b]

            def relu_row(r, _):
                for j in range(D // LANES):
                    sl = pl.ds(j * LANES, LANES)
                    buf[r, sl] = jnp.maximum(buf[r, sl], 0.0)
                return 0

            lax.fori_loop(0, _EDGE_CHUNK, relu_row, 0)
            pltpu.sync_copy(buf, acc_sh.at[dsts[b]], add=True)

        zero = jnp.zeros((LANES,), jnp.float32)

        def zero_row(r, _):
            for j in range(D // LANES):
                buf_a[r, pl.ds(j * LANES, LANES)] = zero
            return 0

        lax.fori_loop(0, _EDGE_CHUNK, zero_row, 0)

        for j in range(zcopies):
            pltpu.sync_copy(buf_a, acc_sh.at[pl.ds(row0 + j * _EDGE_CHUNK,
                                                   _EDGE_CHUNK)])
        if ztail:
            pltpu.sync_copy(buf_a.at[pl.ds(0, ztail)],
                            acc_sh.at[pl.ds(row0 + zcopies * _EDGE_CHUNK,
                                            ztail)])
        if rows_tail:
            @pl.when(s == NS - 1)
            def _():
                pltpu.sync_copy(buf_a.at[pl.ds(0, rows_tail)],
                                acc_sh.at[pl.ds(rows_main * NS, rows_tail)])
        plsc.subcore_barrier()

        stage_issue(0, 0)
        stage_gather(0)
        stage_issue(1, 1)

        def pair_body(g, _):
            j = 2 * g + 1
            stage_gather(1)
            stage_reduce(0)
            stage_issue(j + 1, 0)
            stage_gather(0)
            stage_reduce(1)

            @pl.when(j + 2 < n_chunks)
            def _():
                stage_issue(j + 2, 1)
            return 0

        lax.fori_loop(0, (n_chunks - 1) // 2, pair_body, 0)
        stage_reduce(0)
        plsc.subcore_barrier()
        pltpu.sync_copy(acc_sh.at[pl.ds(row0, rows_main)],
                        out_hbm.at[c, pl.ds(row0, rows_main)])
        if rows_tail:
            @pl.when(s == NS - 1)
            def _():
                pltpu.sync_copy(acc_sh.at[pl.ds(rows_main * NS, rows_tail)],
                                out_hbm.at[c, pl.ds(rows_main * NS,
                                                    rows_tail)])

    return edge_agg


_BN = 2000


def _node_lin1_body(h_ref, p_ref, w1_ref, b1_ref, z1_ref, sums_ref):
    i = pl.program_id(0)
    z = h_ref[...] + p_ref[0] + p_ref[1]
    z1 = jnp.dot(z, w1_ref[...], preferred_element_type=jnp.float32) \
        + b1_ref[...]
    z1_ref[...] = z1

    @pl.when(i == 0)
    def _():
        sums_ref[...] = jnp.zeros_like(sums_ref)

    sums_ref[0, :] += jnp.sum(z1, axis=0)
    sums_ref[1, :] += jnp.sum(z1 * z1, axis=0)


def _node_lin2_body(z1_ref, sums_ref, g_ref, bt_ref, w2_ref, b2_ref, o_ref,
                    *, n_rows, final_relu):
    mu = sums_ref[0, :] / n_rows
    var = sums_ref[1, :] / n_rows - mu * mu
    inv = lax.rsqrt(var + 1e-5)
    zn = (z1_ref[...] - mu) * inv * g_ref[...] + bt_ref[...]
    zn = jnp.maximum(zn, 0.0)
    z2 = jnp.dot(zn, w2_ref[...], preferred_element_type=jnp.float32) \
        + b2_ref[...]
    if final_relu:
        z2 = jnp.maximum(z2, 0.0)
    o_ref[...] = z2


def _node_head_body(z1_ref, sums_ref, g_ref, bt_ref, w2_ref, b2_ref, wd_ref,
                    bd_ref, h_ref, out_ref, *, n_rows):
    mu = sums_ref[0, :] / n_rows
    var = sums_ref[1, :] / n_rows - mu * mu
    inv = lax.rsqrt(var + 1e-5)
    zn = (z1_ref[...] - mu) * inv * g_ref[...] + bt_ref[...]
    zn = jnp.maximum(zn, 0.0)
    z2 = jnp.dot(zn, w2_ref[...], preferred_element_type=jnp.float32) \
        + b2_ref[...]
    h_ref[...] = z2
    out_ref[...] = jnp.dot(z2, wd_ref[...],
                           preferred_element_type=jnp.float32) + bd_ref[...]


def _node_mlp(h, p, w1, b1, g, bt, w2, b2, final_relu, head=None):
    N, D = h.shape
    grid = N // _BN
    full = lambda i: (0, 0)
    vec = lambda i: (0,)
    z1, sums = pl.pallas_call(
        _node_lin1_body,
        grid=(grid,),
        in_specs=[
            pl.BlockSpec((_BN, D), lambda i: (i, 0)),
            pl.BlockSpec((NC, _BN, D), lambda i: (0, i, 0)),
            pl.BlockSpec((D, D), full),
            pl.BlockSpec((D,), vec),
        ],
        out_specs=[
            pl.BlockSpec((_BN, D), lambda i: (i, 0)),
            pl.BlockSpec((8, D), full),
        ],
        out_shape=[
            jax.ShapeDtypeStruct((N, D), jnp.float32),
            jax.ShapeDtypeStruct((8, D), jnp.float32),
        ],
    )(h, p, w1, b1)
    if head is None:
        return pl.pallas_call(
            functools.partial(_node_lin2_body, n_rows=float(N),
                              final_relu=final_relu),
            grid=(grid,),
            in_specs=[
                pl.BlockSpec((_BN, D), lambda i: (i, 0)),
                pl.BlockSpec((8, D), full),
                pl.BlockSpec((D,), vec),
                pl.BlockSpec((D,), vec),
                pl.BlockSpec((D, D), full),
                pl.BlockSpec((D,), vec),
            ],
            out_specs=pl.BlockSpec((_BN, D), lambda i: (i, 0)),
            out_shape=jax.ShapeDtypeStruct((N, D), jnp.float32),
        )(z1, sums, g, bt, w2, b2)
    wd, bd = head
    return pl.pallas_call(
        functools.partial(_node_head_body, n_rows=float(N)),
        grid=(grid,),
        in_specs=[
            pl.BlockSpec((_BN, D), lambda i: (i, 0)),
            pl.BlockSpec((8, D), full),
            pl.BlockSpec((D,), vec),
            pl.BlockSpec((D,), vec),
            pl.BlockSpec((D, D), full),
            pl.BlockSpec((D,), vec),
            pl.BlockSpec((D, 1), full),
            pl.BlockSpec((1,), vec),
        ],
        out_specs=[
            pl.BlockSpec((_BN, D), lambda i: (i, 0)),
            pl.BlockSpec((_BN, 1), lambda i: (i, 0)),
        ],
        out_shape=[
            jax.ShapeDtypeStruct((N, D), jnp.float32),
            jax.ShapeDtypeStruct((N, 1), jnp.float32),
        ],
    )(z1, sums, g, bt, w2, b2, wd, bd)


def kernel(x, edge_index, edge_features, We0, be0, W10, b10, g0, bt0, W20,
           b20, We1, be1, W11, b11, g1, bt1, W21, b21, We2, be2, W12, b12,
           g2, bt2, W22, b22, Wd, bd):
    N, D = x.shape
    E = edge_features.shape[0]
    src = edge_index[0]
    dst = edge_index[1]

    edge_agg = _make_edge_agg(N, D, E)

    e0, e1, e2 = _edge_mlp3(edge_features,
                            [(We0, be0), (We1, be1), (We2, be2)])
    p = edge_agg(x, e0, src, dst)
    h = _node_mlp(x, p, W10, b10, g0, bt0, W20, b20, True)
    p = edge_agg(h, e1, src, dst)
    h = _node_mlp(h, p, W11, b11, g1, bt1, W21, b21, True)
    p = edge_agg(h, e2, src, dst)
    h, out = _node_mlp(h, p, W12, b12, g2, bt2, W22, b22, False,
                       head=(Wd, bd))
    return (out, h)


# speedup vs baseline: 1.4617x; 1.4617x over previous
import functools

import jax
import jax.numpy as jnp
from jax import lax
from jax.experimental import pallas as pl
from jax.experimental.pallas import tpu as pltpu
from jax.experimental.pallas import tpu_sc as plsc

NC = 2
NS = 16
NW = NC * NS
LANES = 16


def _edge_mlp_body(ef_ref, w0_ref, b0_ref, w1_ref, b1_ref, w2_ref, b2_ref,
                   e0_ref, e1_ref, e2_ref):
    ef = ef_ref[...]
    e0_ref[...] = jnp.dot(ef, w0_ref[...],
                          preferred_element_type=jnp.float32) + b0_ref[...]
    e1_ref[...] = jnp.dot(ef, w1_ref[...],
                          preferred_element_type=jnp.float32) + b1_ref[...]
    e2_ref[...] = jnp.dot(ef, w2_ref[...],
                          preferred_element_type=jnp.float32) + b2_ref[...]


def _edge_mlp3(ef, wb):
    E, ED = ef.shape
    D = wb[0][0].shape[1]
    BE = 4000
    grid = E // BE
    full = lambda i: (0, 0)
    vec = lambda i: (0,)
    return pl.pallas_call(
        _edge_mlp_body,
        grid=(grid,),
        in_specs=[
            pl.BlockSpec((BE, ED), lambda i: (i, 0)),
            pl.BlockSpec((ED, D), full), pl.BlockSpec((D,), vec),
            pl.BlockSpec((ED, D), full), pl.BlockSpec((D,), vec),
            pl.BlockSpec((ED, D), full), pl.BlockSpec((D,), vec),
        ],
        out_specs=[pl.BlockSpec((BE, D), lambda i: (i, 0))] * 3,
        out_shape=[jax.ShapeDtypeStruct((E, D), jnp.float32)] * 3,
    )(ef, wb[0][0], wb[0][1], wb[1][0], wb[1][1], wb[2][0], wb[2][1])


_EDGE_CHUNK = 80
_NBUF = 4


def _make_edge_agg(N, D, E):
    e_per_w = E // NW
    n_chunks = e_per_w // _EDGE_CHUNK
    rows_main = (N // NS) // 8 * 8
    rows_tail = N - rows_main * NS
    mesh = plsc.VectorSubcoreMesh(core_axis_name="c", subcore_axis_name="s",
                                  num_cores=NC, num_subcores=NS)
    zcopies = rows_main // _EDGE_CHUNK
    ztail = rows_main - zcopies * _EDGE_CHUNK

    assert (n_chunks - 1) % _NBUF == 0

    @functools.partial(
        pl.kernel,
        out_type=jax.ShapeDtypeStruct((NC, N, D), jnp.float32),
        mesh=mesh,
        scratch_types=(
            [pltpu.VMEM((_EDGE_CHUNK,), jnp.int32)] * _NBUF
            + [pltpu.VMEM((_EDGE_CHUNK,), jnp.int32)] * _NBUF
            + [pltpu.VMEM((_EDGE_CHUNK, D), jnp.float32)] * _NBUF
            + [pltpu.VMEM_SHARED((N, D), jnp.float32)]
            + [pltpu.SemaphoreType.DMA] * (3 * _NBUF)
        ),
    )
    def edge_agg(h_hbm, e_hbm, src_hbm, dst_hbm, out_hbm, *scr):
        srcs = scr[:_NBUF]
        dsts = scr[_NBUF:2 * _NBUF]
        bufs = scr[2 * _NBUF:3 * _NBUF]
        acc_sh = scr[3 * _NBUF]
        semi = scr[3 * _NBUF + 1:3 * _NBUF + 1 + _NBUF]
        seme = scr[3 * _NBUF + 1 + _NBUF:3 * _NBUF + 1 + 2 * _NBUF]
        semg = scr[3 * _NBUF + 1 + 2 * _NBUF:]
        c = lax.axis_index("c")
        s = lax.axis_index("s")
        wid = c * NS + s
        base = wid * e_per_w
        row0 = s * rows_main
        buf_a = bufs[0]

        def stage_issue(j, b):
            off = base + j * _EDGE_CHUNK
            pltpu.async_copy(src_hbm.at[pl.ds(off, _EDGE_CHUNK)],
                             srcs[b], semi[b])
            pltpu.async_copy(dst_hbm.at[pl.ds(off, _EDGE_CHUNK)],
                             dsts[b], semi[b])
            pltpu.async_copy(e_hbm.at[pl.ds(off, _EDGE_CHUNK)],
                             bufs[b], seme[b])

        def stage_gather(b):
            pltpu.make_async_copy(src_hbm.at[pl.ds(0, _EDGE_CHUNK)],
                                  srcs[b], semi[b]).wait()
            pltpu.make_async_copy(dst_hbm.at[pl.ds(0, _EDGE_CHUNK)],
                                  dsts[b], semi[b]).wait()
            pltpu.make_async_copy(e_hbm.at[pl.ds(0, _EDGE_CHUNK)],
                                  bufs[b], seme[b]).wait()
            pltpu.async_copy(h_hbm.at[srcs[b]], bufs[b], semg[b], add=True)

        def stage_reduce(b):
            pltpu.make_async_copy(h_hbm.at[srcs[b]], bufs[b], semg[b]).wait()
            buf = bufs[b]

            def relu_row(r, _):
                for j in range(D // LANES):
                    sl = pl.ds(j * LANES, LANES)
                    buf[r, sl] = jnp.maximum(buf[r, sl], 0.0)
                return 0

            lax.fori_loop(0, _EDGE_CHUNK, relu_row, 0)
            pltpu.sync_copy(buf, acc_sh.at[dsts[b]], add=True)

        zero = jnp.zeros((LANES,), jnp.float32)

        def zero_row(r, _):
            for j in range(D // LANES):
                buf_a[r, pl.ds(j * LANES, LANES)] = zero
            return 0

        lax.fori_loop(0, _EDGE_CHUNK, zero_row, 0)

        for j in range(zcopies):
            pltpu.sync_copy(buf_a, acc_sh.at[pl.ds(row0 + j * _EDGE_CHUNK,
                                                   _EDGE_CHUNK)])
        if ztail:
            pltpu.sync_copy(buf_a.at[pl.ds(0, ztail)],
                            acc_sh.at[pl.ds(row0 + zcopies * _EDGE_CHUNK,
                                            ztail)])
        if rows_tail:
            @pl.when(s == NS - 1)
            def _():
                pltpu.sync_copy(buf_a.at[pl.ds(0, rows_tail)],
                                acc_sh.at[pl.ds(rows_main * NS, rows_tail)])
        plsc.subcore_barrier()

        stage_issue(0, 0)
        stage_gather(0)
        for b in range(1, _NBUF - 1):
            stage_issue(b, b)

        def quad_body(g, _):
            for r in range(_NBUF):
                j = _NBUF * g + r
                stage_gather((r + 1) % _NBUF)
                stage_reduce(r)

                @pl.when(j + _NBUF - 1 < n_chunks)
                def _():
                    stage_issue(j + _NBUF - 1, (r + _NBUF - 1) % _NBUF)
            return 0

        lax.fori_loop(0, (n_chunks - 1) // _NBUF, quad_body, 0)
        stage_reduce((n_chunks - 1) % _NBUF)
        plsc.subcore_barrier()
        pltpu.sync_copy(acc_sh.at[pl.ds(row0, rows_main)],
                        out_hbm.at[c, pl.ds(row0, rows_main)])
        if rows_tail:
            @pl.when(s == NS - 1)
            def _():
                pltpu.sync_copy(acc_sh.at[pl.ds(rows_main * NS, rows_tail)],
                                out_hbm.at[c, pl.ds(rows_main * NS,
                                                    rows_tail)])

    return edge_agg


_BN = 2000


def _node_lin1_body(h_ref, p_ref, w1_ref, b1_ref, z1_ref, sums_ref):
    i = pl.program_id(0)
    z = h_ref[...] + p_ref[0] + p_ref[1]
    z1 = jnp.dot(z, w1_ref[...], preferred_element_type=jnp.float32) \
        + b1_ref[...]
    z1_ref[...] = z1

    @pl.when(i == 0)
    def _():
        sums_ref[...] = jnp.zeros_like(sums_ref)

    sums_ref[0, :] += jnp.sum(z1, axis=0)
    sums_ref[1, :] += jnp.sum(z1 * z1, axis=0)


def _node_lin2_body(z1_ref, sums_ref, g_ref, bt_ref, w2_ref, b2_ref, o_ref,
                    *, n_rows, final_relu):
    mu = sums_ref[0, :] / n_rows
    var = sums_ref[1, :] / n_rows - mu * mu
    inv = lax.rsqrt(var + 1e-5)
    zn = (z1_ref[...] - mu) * inv * g_ref[...] + bt_ref[...]
    zn = jnp.maximum(zn, 0.0)
    z2 = jnp.dot(zn, w2_ref[...], preferred_element_type=jnp.float32) \
        + b2_ref[...]
    if final_relu:
        z2 = jnp.maximum(z2, 0.0)
    o_ref[...] = z2


def _node_head_body(z1_ref, sums_ref, g_ref, bt_ref, w2_ref, b2_ref, wd_ref,
                    bd_ref, h_ref, out_ref, *, n_rows):
    mu = sums_ref[0, :] / n_rows
    var = sums_ref[1, :] / n_rows - mu * mu
    inv = lax.rsqrt(var + 1e-5)
    zn = (z1_ref[...] - mu) * inv * g_ref[...] + bt_ref[...]
    zn = jnp.maximum(zn, 0.0)
    z2 = jnp.dot(zn, w2_ref[...], preferred_element_type=jnp.float32) \
        + b2_ref[...]
    h_ref[...] = z2
    out_ref[...] = jnp.dot(z2, wd_ref[...],
                           preferred_element_type=jnp.float32) + bd_ref[...]


def _node_mlp(h, p, w1, b1, g, bt, w2, b2, final_relu, head=None):
    N, D = h.shape
    grid = N // _BN
    full = lambda i: (0, 0)
    vec = lambda i: (0,)
    z1, sums = pl.pallas_call(
        _node_lin1_body,
        grid=(grid,),
        in_specs=[
            pl.BlockSpec((_BN, D), lambda i: (i, 0)),
            pl.BlockSpec((NC, _BN, D), lambda i: (0, i, 0)),
            pl.BlockSpec((D, D), full),
            pl.BlockSpec((D,), vec),
        ],
        out_specs=[
            pl.BlockSpec((_BN, D), lambda i: (i, 0)),
            pl.BlockSpec((8, D), full),
        ],
        out_shape=[
            jax.ShapeDtypeStruct((N, D), jnp.float32),
            jax.ShapeDtypeStruct((8, D), jnp.float32),
        ],
    )(h, p, w1, b1)
    if head is None:
        return pl.pallas_call(
            functools.partial(_node_lin2_body, n_rows=float(N),
                              final_relu=final_relu),
            grid=(grid,),
            in_specs=[
                pl.BlockSpec((_BN, D), lambda i: (i, 0)),
                pl.BlockSpec((8, D), full),
                pl.BlockSpec((D,), vec),
                pl.BlockSpec((D,), vec),
                pl.BlockSpec((D, D), full),
                pl.BlockSpec((D,), vec),
            ],
            out_specs=pl.BlockSpec((_BN, D), lambda i: (i, 0)),
            out_shape=jax.ShapeDtypeStruct((N, D), jnp.float32),
        )(z1, sums, g, bt, w2, b2)
    wd, bd = head
    return pl.pallas_call(
        functools.partial(_node_head_body, n_rows=float(N)),
        grid=(grid,),
        in_specs=[
            pl.BlockSpec((_BN, D), lambda i: (i, 0)),
            pl.BlockSpec((8, D), full),
            pl.BlockSpec((D,), vec),
            pl.BlockSpec((D,), vec),
            pl.BlockSpec((D, D), full),
            pl.BlockSpec((D,), vec),
            pl.BlockSpec((D, 1), full),
            pl.BlockSpec((1,), vec),
        ],
        out_specs=[
            pl.BlockSpec((_BN, D), lambda i: (i, 0)),
            pl.BlockSpec((_BN, 1), lambda i: (i, 0)),
        ],
        out_shape=[
            jax.ShapeDtypeStruct((N, D), jnp.float32),
            jax.ShapeDtypeStruct((N, 1), jnp.float32),
        ],
    )(z1, sums, g, bt, w2, b2, wd, bd)


def kernel(x, edge_index, edge_features, We0, be0, W10, b10, g0, bt0, W20,
           b20, We1, be1, W11, b11, g1, bt1, W21, b21, We2, be2, W12, b12,
           g2, bt2, W22, b22, Wd, bd):
    N, D = x.shape
    E = edge_features.shape[0]
    src = edge_index[0]
    dst = edge_index[1]

    edge_agg = _make_edge_agg(N, D, E)

    e0, e1, e2 = _edge_mlp3(edge_features,
                            [(We0, be0), (We1, be1), (We2, be2)])
    p = edge_agg(x, e0, src, dst)
    h = _node_mlp(x, p, W10, b10, g0, bt0, W20, b20, True)
    p = edge_agg(h, e1, src, dst)
    h = _node_mlp(h, p, W11, b11, g1, bt1, W21, b21, True)
    p = edge_agg(h, e2, src, dst)
    h, out = _node_mlp(h, p, W12, b12, g2, bt2, W22, b22, False,
                       head=(Wd, bd))
    return (out, h)


# speedup vs baseline: 1.4884x; 1.0183x over previous
import functools

import jax
import jax.numpy as jnp
from jax import lax
from jax.experimental import pallas as pl
from jax.experimental.pallas import tpu as pltpu
from jax.experimental.pallas import tpu_sc as plsc

NC = 2
NS = 16
NW = NC * NS
LANES = 16


def _edge_mlp_body(ef_ref, w0_ref, b0_ref, w1_ref, b1_ref, w2_ref, b2_ref,
                   e0_ref, e1_ref, e2_ref):
    ef = ef_ref[...]
    e0_ref[...] = jnp.dot(ef, w0_ref[...],
                          preferred_element_type=jnp.float32) + b0_ref[...]
    e1_ref[...] = jnp.dot(ef, w1_ref[...],
                          preferred_element_type=jnp.float32) + b1_ref[...]
    e2_ref[...] = jnp.dot(ef, w2_ref[...],
                          preferred_element_type=jnp.float32) + b2_ref[...]


def _edge_mlp3(ef, wb):
    E, ED = ef.shape
    D = wb[0][0].shape[1]
    BE = 4000
    grid = E // BE
    full = lambda i: (0, 0)
    vec = lambda i: (0,)
    return pl.pallas_call(
        _edge_mlp_body,
        grid=(grid,),
        in_specs=[
            pl.BlockSpec((BE, ED), lambda i: (i, 0)),
            pl.BlockSpec((ED, D), full), pl.BlockSpec((D,), vec),
            pl.BlockSpec((ED, D), full), pl.BlockSpec((D,), vec),
            pl.BlockSpec((ED, D), full), pl.BlockSpec((D,), vec),
        ],
        out_specs=[pl.BlockSpec((BE, D), lambda i: (i, 0))] * 3,
        out_shape=[jax.ShapeDtypeStruct((E, D), jnp.float32)] * 3,
    )(ef, wb[0][0], wb[0][1], wb[1][0], wb[1][1], wb[2][0], wb[2][1])


_EDGE_CHUNK = 80
_NBUF = 4


def _make_edge_agg(N, D, E):
    e_per_w = E // NW
    n_chunks = e_per_w // _EDGE_CHUNK
    rows_main = (N // NS) // 8 * 8
    rows_tail = N - rows_main * NS
    mesh = plsc.VectorSubcoreMesh(core_axis_name="c", subcore_axis_name="s",
                                  num_cores=NC, num_subcores=NS)
    zcopies = rows_main // _EDGE_CHUNK
    ztail = rows_main - zcopies * _EDGE_CHUNK

    assert (n_chunks - 1) % _NBUF == 0

    @functools.partial(
        pl.kernel,
        out_type=jax.ShapeDtypeStruct((NC, N, D), jnp.float32),
        mesh=mesh,
        scratch_types=(
            [pltpu.VMEM((_EDGE_CHUNK,), jnp.int32)] * _NBUF
            + [pltpu.VMEM((_EDGE_CHUNK,), jnp.int32)] * _NBUF
            + [pltpu.VMEM((_EDGE_CHUNK, D), jnp.float32)] * _NBUF
            + [pltpu.VMEM_SHARED((N, D), jnp.float32)]
            + [pltpu.SemaphoreType.DMA] * (3 * _NBUF)
        ),
    )
    def edge_agg(h_hbm, e_hbm, src_hbm, dst_hbm, out_hbm, *scr):
        srcs = scr[:_NBUF]
        dsts = scr[_NBUF:2 * _NBUF]
        bufs = scr[2 * _NBUF:3 * _NBUF]
        acc_sh = scr[3 * _NBUF]
        semi = scr[3 * _NBUF + 1:3 * _NBUF + 1 + _NBUF]
        seme = scr[3 * _NBUF + 1 + _NBUF:3 * _NBUF + 1 + 2 * _NBUF]
        semg = scr[3 * _NBUF + 1 + 2 * _NBUF:]
        c = lax.axis_index("c")
        s = lax.axis_index("s")
        wid = c * NS + s
        base = wid * e_per_w
        row0 = s * rows_main
        buf_a = bufs[0]

        def stage_issue(j, b):
            off = base + j * _EDGE_CHUNK
            pltpu.async_copy(src_hbm.at[pl.ds(off, _EDGE_CHUNK)],
                             srcs[b], semi[b])
            pltpu.async_copy(dst_hbm.at[pl.ds(off, _EDGE_CHUNK)],
                             dsts[b], semi[b])
            pltpu.async_copy(e_hbm.at[pl.ds(off, _EDGE_CHUNK)],
                             bufs[b], seme[b])

        def stage_gather(b):
            pltpu.make_async_copy(src_hbm.at[pl.ds(0, _EDGE_CHUNK)],
                                  srcs[b], semi[b]).wait()
            pltpu.make_async_copy(dst_hbm.at[pl.ds(0, _EDGE_CHUNK)],
                                  dsts[b], semi[b]).wait()
            pltpu.make_async_copy(e_hbm.at[pl.ds(0, _EDGE_CHUNK)],
                                  bufs[b], seme[b]).wait()
            pltpu.async_copy(h_hbm.at[srcs[b]], bufs[b], semg[b], add=True)

        def stage_reduce(b):
            pltpu.make_async_copy(h_hbm.at[srcs[b]], bufs[b], semg[b]).wait()
            buf = bufs[b]

            def relu_row(r, _):
                for j in range(D // LANES):
                    sl = pl.ds(j * LANES, LANES)
                    buf[r, sl] = jnp.maximum(buf[r, sl], 0.0)
                return 0

            lax.fori_loop(0, _EDGE_CHUNK, relu_row, 0)
            pltpu.sync_copy(buf, acc_sh.at[dsts[b]], add=True)

        zero = jnp.zeros((LANES,), jnp.float32)

        def zero_row(r, _):
            for j in range(D // LANES):
                buf_a[r, pl.ds(j * LANES, LANES)] = zero
            return 0

        lax.fori_loop(0, _EDGE_CHUNK, zero_row, 0)

        for j in range(zcopies):
            pltpu.sync_copy(buf_a, acc_sh.at[pl.ds(row0 + j * _EDGE_CHUNK,
                                                   _EDGE_CHUNK)])
        if ztail:
            pltpu.sync_copy(buf_a.at[pl.ds(0, ztail)],
                            acc_sh.at[pl.ds(row0 + zcopies * _EDGE_CHUNK,
                                            ztail)])
        if rows_tail:
            @pl.when(s == NS - 1)
            def _():
                pltpu.sync_copy(buf_a.at[pl.ds(0, rows_tail)],
                                acc_sh.at[pl.ds(rows_main * NS, rows_tail)])
        plsc.subcore_barrier()

        stage_issue(0, 0)
        stage_gather(0)
        for b in range(1, _NBUF - 1):
            stage_issue(b, b)

        def quad_body(g, _):
            for r in range(_NBUF):
                j = _NBUF * g + r
                stage_gather((r + 1) % _NBUF)
                stage_reduce(r)

                @pl.when(j + _NBUF - 1 < n_chunks)
                def _():
                    stage_issue(j + _NBUF - 1, (r + _NBUF - 1) % _NBUF)
            return 0

        lax.fori_loop(0, (n_chunks - 1) // _NBUF, quad_body, 0)
        stage_reduce((n_chunks - 1) % _NBUF)
        plsc.subcore_barrier()
        pltpu.sync_copy(acc_sh.at[pl.ds(row0, rows_main)],
                        out_hbm.at[c, pl.ds(row0, rows_main)])
        if rows_tail:
            @pl.when(s == NS - 1)
            def _():
                pltpu.sync_copy(acc_sh.at[pl.ds(rows_main * NS, rows_tail)],
                                out_hbm.at[c, pl.ds(rows_main * NS,
                                                    rows_tail)])

    return edge_agg


_BN = 2000


def _node_body(h_ref, p_ref, w1_ref, b1_ref, g_ref, bt_ref, w2_ref, b2_ref,
               *rest, n_rows, final_relu, head):
    if head:
        wd_ref, bd_ref, o_ref, out_ref, z1_scr, sums_scr = rest
    else:
        o_ref, z1_scr, sums_scr = rest
    ph = pl.program_id(0)
    i = pl.program_id(1)
    rows = pl.ds(i * _BN, _BN)

    @pl.when(ph == 0)
    def _():
        z = h_ref[...] + p_ref[0] + p_ref[1]
        z1 = jnp.dot(z, w1_ref[...], preferred_element_type=jnp.float32) \
            + b1_ref[...]
        z1_scr[rows, :] = z1

        @pl.when(i == 0)
        def _():
            sums_scr[...] = jnp.zeros_like(sums_scr)

        sums_scr[0, :] += jnp.sum(z1, axis=0)
        sums_scr[1, :] += jnp.sum(z1 * z1, axis=0)

    @pl.when(ph == 1)
    def _():
        mu = sums_scr[0, :] / n_rows
        var = sums_scr[1, :] / n_rows - mu * mu
        inv = lax.rsqrt(var + 1e-5)
        zn = (z1_scr[rows, :] - mu) * inv * g_ref[...] + bt_ref[...]
        zn = jnp.maximum(zn, 0.0)
        z2 = jnp.dot(zn, w2_ref[...], preferred_element_type=jnp.float32) \
            + b2_ref[...]
        if final_relu:
            z2 = jnp.maximum(z2, 0.0)
        o_ref[...] = z2
        if head:
            out_ref[...] = jnp.dot(z2, wd_ref[...],
                                   preferred_element_type=jnp.float32) \
                + bd_ref[...]


def _node_mlp(h, p, w1, b1, g, bt, w2, b2, final_relu, head=None):
    N, D = h.shape
    grid_n = N // _BN
    full = lambda ph, i: (0, 0)
    vec = lambda ph, i: (0,)
    blk = lambda ph, i: (i * ph, 0)
    in_specs = [
        pl.BlockSpec((_BN, D), lambda ph, i: (i * (1 - ph), 0)),
        pl.BlockSpec((NC, _BN, D), lambda ph, i: (0, i * (1 - ph), 0)),
        pl.BlockSpec((D, D), full),
        pl.BlockSpec((D,), vec),
        pl.BlockSpec((D,), vec),
        pl.BlockSpec((D,), vec),
        pl.BlockSpec((D, D), full),
        pl.BlockSpec((D,), vec),
    ]
    args = [h, p, w1, b1, g, bt, w2, b2]
    if head is None:
        out_specs = pl.BlockSpec((_BN, D), blk)
        out_shape = jax.ShapeDtypeStruct((N, D), jnp.float32)
    else:
        in_specs += [pl.BlockSpec((D, 1), full), pl.BlockSpec((1,), vec)]
        args += list(head)
        out_specs = [pl.BlockSpec((_BN, D), blk), pl.BlockSpec((_BN, 1), blk)]
        out_shape = [jax.ShapeDtypeStruct((N, D), jnp.float32),
                     jax.ShapeDtypeStruct((N, 1), jnp.float32)]
    return pl.pallas_call(
        functools.partial(_node_body, n_rows=float(N), final_relu=final_relu,
                          head=head is not None),
        grid=(2, grid_n),
        in_specs=in_specs,
        out_specs=out_specs,
        out_shape=out_shape,
        scratch_shapes=[
            pltpu.VMEM((N, D), jnp.float32),
            pltpu.VMEM((8, D), jnp.float32),
        ],
    )(*args)


def kernel(x, edge_index, edge_features, We0, be0, W10, b10, g0, bt0, W20,
           b20, We1, be1, W11, b11, g1, bt1, W21, b21, We2, be2, W12, b12,
           g2, bt2, W22, b22, Wd, bd):
    N, D = x.shape
    E = edge_features.shape[0]
    src = edge_index[0]
    dst = edge_index[1]

    edge_agg = _make_edge_agg(N, D, E)

    e0, e1, e2 = _edge_mlp3(edge_features,
                            [(We0, be0), (We1, be1), (We2, be2)])
    p = edge_agg(x, e0, src, dst)
    h = _node_mlp(x, p, W10, b10, g0, bt0, W20, b20, True)
    p = edge_agg(h, e1, src, dst)
    h = _node_mlp(h, p, W11, b11, g1, bt1, W21, b21, True)
    p = edge_agg(h, e2, src, dst)
    h, out = _node_mlp(h, p, W12, b12, g2, bt2, W22, b22, False,
                       head=(Wd, bd))
    return (out, h)
